# baseline (speedup 1.0000x reference)
import functools

import jax
import jax.numpy as jnp
from jax import lax
from jax.experimental import pallas as pl
from jax.experimental.pallas import tpu as pltpu
from jax.experimental.pallas import tpu_sc as plsc

LANES = 16
NC, NS = 2, 16
NW = NC * NS
CHUNK = 128
WAVE = 4 * CHUNK
HALF = WAVE // 2


def _rsqrt(v):
    bits = lax.bitcast_convert_type(v, jnp.int32)
    y = lax.bitcast_convert_type(jnp.int32(0x5F3759DF) - (bits >> 1), jnp.float32)
    for _ in range(3):
        y = y * (jnp.float32(1.5) - jnp.float32(0.5) * v * y * y)
    return y


def _make_sc_kernel(n_tokens, vocab, emb, n_pos):
    per_w = n_tokens // NW
    n_waves = per_w // WAVE
    groups_per_half = HALF // LANES
    nsub = emb // LANES
    assert per_w % WAVE == 0
    mesh = plsc.VectorSubcoreMesh(core_axis_name="c", subcore_axis_name="s")

    @functools.partial(
        pl.kernel,
        out_type=jax.ShapeDtypeStruct((n_tokens * emb,), jnp.float32),
        mesh=mesh,
        compiler_params=pltpu.CompilerParams(
            use_tc_tiling_on_sc=False, needs_layout_passes=False),
        scratch_types=[
            pltpu.VMEM((WAVE,), jnp.int32),
            pltpu.VMEM((WAVE,), jnp.int32),
            pltpu.VMEM((WAVE,), jnp.int32),
            pltpu.VMEM((WAVE,), jnp.int32),
            pltpu.VMEM((WAVE, emb), jnp.float32),
            pltpu.VMEM((HALF * emb,), jnp.float32),
            pltpu.VMEM((HALF * emb,), jnp.float32),
            pltpu.VMEM((n_pos, emb), jnp.float32),
            pltpu.VMEM((emb,), jnp.float32),
            pltpu.VMEM((emb,), jnp.float32),
            pltpu.SemaphoreType.DMA,
            pltpu.SemaphoreType.DMA,
            pltpu.SemaphoreType.DMA,
            pltpu.SemaphoreType.DMA,
            pltpu.SemaphoreType.DMA,
            pltpu.SemaphoreType.DMA,
        ],
    )
    def sc_kernel(src_h, seg_h, word_h, pos_h, g_h, b_h, out_h,
                  idxn, segb, pidxn, pidxc, wbuf, obuf0, obuf1, posv, gv, bv,
                  g0, g1, g2, g3, s0, s1):
        wid = lax.axis_index("s") * NC + lax.axis_index("c")
        pltpu.sync_copy(pos_h, posv)
        pltpu.sync_copy(g_h, gv)
        pltpu.sync_copy(b_h, bv)
        base0 = wid * per_w
        gs = [gv[pl.ds(LANES * k, LANES)] for k in range(nsub)]
        bs = [bv[pl.ds(LANES * k, LANES)] for k in range(nsub)]
        inv_e = jnp.float32(1.0 / emb)
        iota = lax.iota(jnp.int32, LANES)
        gsems = (g0, g1, g2, g3)
        obufs = (obuf0, obuf1)
        ssems = (s0, s1)

        def gather_q(b):
            return pltpu.make_async_copy(
                word_h.at[idxn.at[pl.ds(b * CHUNK, CHUNK)]],
                wbuf.at[pl.ds(b * CHUNK, CHUNK)],
                gsems[b])

        def store_h(h, w):
            base = (base0 + w * WAVE + h * HALF) * emb
            return pltpu.make_async_copy(
                obufs[h], out_h.at[pl.ds(base, HALF * emb)], ssems[h])

        def stage(w):
            base = base0 + w * WAVE
            pltpu.sync_copy(src_h.at[pl.ds(base, WAVE)], idxn)
            pltpu.sync_copy(seg_h.at[pl.ds(base, WAVE)], segb)

            def pix_body(j):
                s = segb[pl.ds(j * LANES, LANES)]
                q = (s.astype(jnp.float32) * jnp.float32(1.0 / 10000.0)).astype(jnp.int32)
                r = s - q * 10000
                q = jnp.where(r >= 10000, q + 1, q)
                q = jnp.where(r < 0, q - 1, q)
                pidxn[pl.ds(j * LANES, LANES)] = q

            plsc.parallel_loop(0, WAVE // LANES, 1)(pix_body)

        def compute_half(h):
            obuf = obufs[h]

            def group_body(g):
                lrow = g * LANES
                wrow = h * HALF + lrow
                rowv = iota + wrow
                pvec = pidxc[pl.ds(wrow, LANES)]
                acc = [jnp.zeros((LANES,), jnp.float32) for _ in range(4)]
                acc2 = [jnp.zeros((LANES,), jnp.float32) for _ in range(4)]
                for d in range(emb):
                    dskew = (iota + d) & (emb - 1)
                    x = (plsc.load_gather(wbuf, [rowv, dskew])
                         + plsc.load_gather(posv, [pvec, dskew]))
                    acc[d % 4] = acc[d % 4] + x
                    acc2[d % 4] = acc2[d % 4] + x * x
                sumv = (acc[0] + acc[1]) + (acc[2] + acc[3])
                sumsqv = (acc2[0] + acc2[1]) + (acc2[2] + acc2[3])
                meanvec = sumv * inv_e
                varvec = sumsqv * inv_e - meanvec * meanvec
                rstdvec = _rsqrt(varvec + jnp.float32(1e-6))
                for i in range(LANES):
                    t = lrow + i
                    pt = pvec[i]
                    msp = jnp.full((LANES,), meanvec[i], jnp.float32)
                    rsp = jnp.full((LANES,), rstdvec[i], jnp.float32)
                    for k in range(nsub):
                        xk = (wbuf[wrow + i, pl.ds(LANES * k, LANES)]
                              + posv[pt, pl.ds(LANES * k, LANES)])
                        obuf[pl.ds(t * emb + LANES * k, LANES)] = (
                            (xk - msp) * rsp * gs[k] + bs[k])

            plsc.parallel_loop(0, groups_per_half, 1)(group_body)

        def copy_pidx():
            def cp(j):
                pidxc[pl.ds(j * LANES, LANES)] = pidxn[pl.ds(j * LANES, LANES)]
            plsc.parallel_loop(0, WAVE // LANES, 1)(cp)

        stage(0)
        for b in range(4):
            gather_q(b).start()

        def wave_body(w, _):
            copy_pidx()

            gather_q(0).wait()
            gather_q(1).wait()

            @pl.when(w >= 1)
            def _():
                store_h(0, w - 1).wait()

            compute_half(0)
            store_h(0, w).start()

            gather_q(2).wait()
            gather_q(3).wait()

            @pl.when(w < n_waves - 1)
            def _():
                stage(w + 1)
                gather_q(0).start()
                gather_q(1).start()

            @pl.when(w >= 1)
            def _():
                store_h(1, w - 1).wait()

            compute_half(1)
            store_h(1, w).start()

            @pl.when(w < n_waves - 1)
            def _():
                gather_q(2).start()
                gather_q(3).start()

            return 0

        lax.fori_loop(0, n_waves, wave_body, 0)
        store_h(0, n_waves - 1).wait()
        store_h(1, n_waves - 1).wait()

    return sc_kernel


def kernel(src, seg, word_table, pos_table, gamma, beta):
    b, l = src.shape
    vocab, emb = word_table.shape
    n_pos = pos_table.shape[0]
    n = b * l
    flat_src = src.reshape(n).astype(jnp.int32)
    flat_seg = seg.reshape(n).astype(jnp.int32)
    sc = _make_sc_kernel(n, vocab, emb, n_pos)
    out = sc(flat_src, flat_seg, word_table, pos_table, gamma, beta)
    return out.reshape(b, l, emb)


# speedup vs baseline: 1.0599x; 1.0599x over previous
import functools

import jax
import jax.numpy as jnp
from jax import lax
from jax.experimental import pallas as pl
from jax.experimental.pallas import tpu as pltpu
from jax.experimental.pallas import tpu_sc as plsc

LANES = 16
NC, NS = 2, 16
NW = NC * NS
CHUNK = 128
WAVE = 4 * CHUNK
HALF = WAVE // 2


def _rsqrt(v):
    bits = lax.bitcast_convert_type(v, jnp.int32)
    y = lax.bitcast_convert_type(jnp.int32(0x5F3759DF) - (bits >> 1), jnp.float32)
    for _ in range(3):
        y = y * (jnp.float32(1.5) - jnp.float32(0.5) * v * y * y)
    return y


def _make_sc_kernel(n_tokens, vocab, emb, n_pos):
    per_w = n_tokens // NW
    n_waves = per_w // WAVE
    groups_per_half = HALF // LANES
    nsub = emb // LANES
    assert per_w % WAVE == 0
    mesh = plsc.VectorSubcoreMesh(core_axis_name="c", subcore_axis_name="s")

    @functools.partial(
        pl.kernel,
        out_type=jax.ShapeDtypeStruct((n_tokens * emb,), jnp.float32),
        mesh=mesh,
        compiler_params=pltpu.CompilerParams(
            use_tc_tiling_on_sc=False, needs_layout_passes=False),
        scratch_types=[
            pltpu.VMEM((WAVE,), jnp.int32),
            pltpu.VMEM((WAVE,), jnp.int32),
            pltpu.VMEM((WAVE,), jnp.int32),
            pltpu.VMEM((WAVE,), jnp.int32),
            pltpu.VMEM((WAVE, emb), jnp.float32),
            pltpu.VMEM((HALF * emb,), jnp.float32),
            pltpu.VMEM((HALF * emb,), jnp.float32),
            pltpu.VMEM((n_pos, emb), jnp.float32),
            pltpu.VMEM((emb,), jnp.float32),
            pltpu.VMEM((emb,), jnp.float32),
            pltpu.SemaphoreType.DMA,
            pltpu.SemaphoreType.DMA,
            pltpu.SemaphoreType.DMA,
            pltpu.SemaphoreType.DMA,
            pltpu.SemaphoreType.DMA,
            pltpu.SemaphoreType.DMA,
        ],
    )
    def sc_kernel(src_h, seg_h, word_h, pos_h, g_h, b_h, out_h,
                  idxn, segb, pidxn, pidxc, wbuf, obuf0, obuf1, posv, gv, bv,
                  g0, g1, g2, g3, s0, s1):
        wid = lax.axis_index("s") * NC + lax.axis_index("c")
        pltpu.sync_copy(pos_h, posv)
        pltpu.sync_copy(g_h, gv)
        pltpu.sync_copy(b_h, bv)
        base0 = wid * per_w
        gs = [gv[pl.ds(LANES * k, LANES)] for k in range(nsub)]
        bs = [bv[pl.ds(LANES * k, LANES)] for k in range(nsub)]
        inv_e = jnp.float32(1.0 / emb)
        iota = lax.iota(jnp.int32, LANES)
        gsems = (g0, g1, g2, g3)
        obufs = (obuf0, obuf1)
        ssems = (s0, s1)

        def gather_q(b):
            return pltpu.make_async_copy(
                word_h.at[idxn.at[pl.ds(b * CHUNK, CHUNK)]],
                wbuf.at[pl.ds(b * CHUNK, CHUNK)],
                gsems[b])

        def store_h(h, w):
            base = (base0 + w * WAVE + h * HALF) * emb
            return pltpu.make_async_copy(
                obufs[h], out_h.at[pl.ds(base, HALF * emb)], ssems[h])

        def stage(w):
            base = base0 + w * WAVE
            pltpu.sync_copy(src_h.at[pl.ds(base, WAVE)], idxn)
            pltpu.sync_copy(seg_h.at[pl.ds(base, WAVE)], segb)

            def pix_body(j):
                s = segb[pl.ds(j * LANES, LANES)]
                q = (s.astype(jnp.float32) * jnp.float32(1.0 / 10000.0)).astype(jnp.int32)
                r = s - q * 10000
                q = jnp.where(r >= 10000, q + 1, q)
                q = jnp.where(r < 0, q - 1, q)
                pidxn[pl.ds(j * LANES, LANES)] = q

            plsc.parallel_loop(0, WAVE // LANES, 1)(pix_body)

        def compute_half(h):
            obuf = obufs[h]

            def group_body(g):
                lrow = g * LANES
                wrow = h * HALF + lrow
                rowv = iota + wrow
                oflat = (iota + lrow) * emb
                pvec = pidxc[pl.ds(wrow, LANES)]
                acc = [jnp.zeros((LANES,), jnp.float32) for _ in range(4)]
                acc2 = [jnp.zeros((LANES,), jnp.float32) for _ in range(4)]
                for d in range(emb):
                    dskew = (iota * 9 + d) & (emb - 1)
                    x = (plsc.load_gather(wbuf, [rowv, dskew])
                         + plsc.load_gather(posv, [pvec, dskew]))
                    acc[d % 4] = acc[d % 4] + x
                    acc2[d % 4] = acc2[d % 4] + x * x
                    plsc.store_scatter(obuf, [oflat + dskew], x)
                sumv = (acc[0] + acc[1]) + (acc[2] + acc[3])
                sumsqv = (acc2[0] + acc2[1]) + (acc2[2] + acc2[3])
                meanvec = sumv * inv_e
                varvec = sumsqv * inv_e - meanvec * meanvec
                rstdvec = _rsqrt(varvec + jnp.float32(1e-6))
                for i in range(LANES):
                    t = lrow + i
                    msp = jnp.full((LANES,), meanvec[i], jnp.float32)
                    rsp = jnp.full((LANES,), rstdvec[i], jnp.float32)
                    for k in range(nsub):
                        sl = pl.ds(t * emb + LANES * k, LANES)
                        xk = obuf[sl]
                        obuf[sl] = (xk - msp) * rsp * gs[k] + bs[k]

            plsc.parallel_loop(0, groups_per_half, 1)(group_body)

        def copy_pidx():
            def cp(j):
                pidxc[pl.ds(j * LANES, LANES)] = pidxn[pl.ds(j * LANES, LANES)]
            plsc.parallel_loop(0, WAVE // LANES, 1)(cp)

        stage(0)
        for b in range(4):
            gather_q(b).start()

        def wave_body(w, _):
            copy_pidx()

            gather_q(0).wait()
            gather_q(1).wait()

            @pl.when(w >= 1)
            def _():
                store_h(0, w - 1).wait()

            compute_half(0)
            store_h(0, w).start()

            gather_q(2).wait()
            gather_q(3).wait()

            @pl.when(w < n_waves - 1)
            def _():
                stage(w + 1)
                gather_q(0).start()
                gather_q(1).start()

            @pl.when(w >= 1)
            def _():
                store_h(1, w - 1).wait()

            compute_half(1)
            store_h(1, w).start()

            @pl.when(w < n_waves - 1)
            def _():
                gather_q(2).start()
                gather_q(3).start()

            return 0

        lax.fori_loop(0, n_waves, wave_body, 0)
        store_h(0, n_waves - 1).wait()
        store_h(1, n_waves - 1).wait()

    return sc_kernel


def kernel(src, seg, word_table, pos_table, gamma, beta):
    b, l = src.shape
    vocab, emb = word_table.shape
    n_pos = pos_table.shape[0]
    n = b * l
    flat_src = src.reshape(n).astype(jnp.int32)
    flat_seg = seg.reshape(n).astype(jnp.int32)
    sc = _make_sc_kernel(n, vocab, emb, n_pos)
    out = sc(flat_src, flat_seg, word_table, pos_table, gamma, beta)
    return out.reshape(b, l, emb)


# speedup vs baseline: 1.2787x; 1.2064x over previous
import functools

import jax
import jax.numpy as jnp
from jax import lax
from jax.experimental import pallas as pl
from jax.experimental.pallas import tpu as pltpu
from jax.experimental.pallas import tpu_sc as plsc

LANES = 16
NC, NS = 2, 16
NW = NC * NS
CHUNK = 128
WAVE = 4 * CHUNK
HALF = WAVE // 2
TCP = 128


def _make_sc_gather(n_tokens, vocab, emb, n_pos):
    per_w = n_tokens // NW
    n_waves = per_w // WAVE
    groups_per_half = HALF // LANES
    nsub = emb // LANES
    assert per_w % WAVE == 0
    mesh = plsc.VectorSubcoreMesh(core_axis_name="c", subcore_axis_name="s")

    @functools.partial(
        pl.kernel,
        out_type=jax.ShapeDtypeStruct((n_tokens * emb,), jnp.float32),
        mesh=mesh,
        compiler_params=pltpu.CompilerParams(
            use_tc_tiling_on_sc=False, needs_layout_passes=False),
        scratch_types=[
            pltpu.VMEM((WAVE,), jnp.int32),
            pltpu.VMEM((WAVE,), jnp.int32),
            pltpu.VMEM((WAVE,), jnp.int32),
            pltpu.VMEM((WAVE,), jnp.int32),
            pltpu.VMEM((WAVE, emb), jnp.float32),
            pltpu.VMEM((HALF * emb,), jnp.float32),
            pltpu.VMEM((HALF * emb,), jnp.float32),
            pltpu.VMEM((n_pos, emb), jnp.float32),
            pltpu.SemaphoreType.DMA,
            pltpu.SemaphoreType.DMA,
            pltpu.SemaphoreType.DMA,
            pltpu.SemaphoreType.DMA,
            pltpu.SemaphoreType.DMA,
            pltpu.SemaphoreType.DMA,
        ],
    )
    def sc_kernel(src_h, seg_h, word_h, pos_h, out_h,
                  idxn, segb, pidxn, pidxc, wbuf, obuf0, obuf1, posv,
                  g0, g1, g2, g3, s0, s1):
        wid = lax.axis_index("s") * NC + lax.axis_index("c")
        pltpu.sync_copy(pos_h, posv)
        base0 = wid * per_w
        iota = lax.iota(jnp.int32, LANES)
        gsems = (g0, g1, g2, g3)
        obufs = (obuf0, obuf1)
        ssems = (s0, s1)

        def gather_q(b):
            return pltpu.make_async_copy(
                word_h.at[idxn.at[pl.ds(b * CHUNK, CHUNK)]],
                wbuf.at[pl.ds(b * CHUNK, CHUNK)],
                gsems[b])

        def store_h(h, w):
            base = (base0 + w * WAVE + h * HALF) * emb
            return pltpu.make_async_copy(
                obufs[h], out_h.at[pl.ds(base, HALF * emb)], ssems[h])

        def stage(w):
            base = base0 + w * WAVE
            pltpu.sync_copy(src_h.at[pl.ds(base, WAVE)], idxn)
            pltpu.sync_copy(seg_h.at[pl.ds(base, WAVE)], segb)

            def pix_body(j):
                s = segb[pl.ds(j * LANES, LANES)]
                q = (s.astype(jnp.float32) * jnp.float32(1.0 / 10000.0)).astype(jnp.int32)
                r = s - q * 10000
                q = jnp.where(r >= 10000, q + 1, q)
                q = jnp.where(r < 0, q - 1, q)
                pidxn[pl.ds(j * LANES, LANES)] = q

            plsc.parallel_loop(0, WAVE // LANES, 1)(pix_body)

        def compute_half(h):
            obuf = obufs[h]

            def group_body(g):
                lrow = g * LANES
                wrow = h * HALF + lrow
                pvec = pidxc[pl.ds(wrow, LANES)]
                for i in range(LANES):
                    t = lrow + i
                    pt = pvec[i]
                    for k in range(nsub):
                        obuf[pl.ds(t * emb + LANES * k, LANES)] = (
                            wbuf[wrow + i, pl.ds(LANES * k, LANES)]
                            + posv[pt, pl.ds(LANES * k, LANES)])

            plsc.parallel_loop(0, groups_per_half, 1)(group_body)

        def copy_pidx():
            def cp(j):
                pidxc[pl.ds(j * LANES, LANES)] = pidxn[pl.ds(j * LANES, LANES)]
            plsc.parallel_loop(0, WAVE // LANES, 1)(cp)

        stage(0)
        for b in range(4):
            gather_q(b).start()

        def wave_body(w, _):
            copy_pidx()

            gather_q(0).wait()
            gather_q(1).wait()

            @pl.when(w >= 1)
            def _():
                store_h(0, w - 1).wait()

            compute_half(0)
            store_h(0, w).start()

            gather_q(2).wait()
            gather_q(3).wait()

            @pl.when(w < n_waves - 1)
            def _():
                stage(w + 1)
                gather_q(0).start()
                gather_q(1).start()

            @pl.when(w >= 1)
            def _():
                store_h(1, w - 1).wait()

            compute_half(1)
            store_h(1, w).start()

            @pl.when(w < n_waves - 1)
            def _():
                gather_q(2).start()
                gather_q(3).start()

            return 0

        lax.fori_loop(0, n_waves, wave_body, 0)
        store_h(0, n_waves - 1).wait()
        store_h(1, n_waves - 1).wait()

    return sc_kernel


def _make_tc_layernorm(n_rows, emb):
    blk = 1024
    assert n_rows % blk == 0
    inv_e = float(1.0 / emb)

    def body(x_ref, sm_ref, g_ref, b_ref, o_ref):
        x = x_ref[...]
        sm = sm_ref[...]
        s1 = jax.lax.dot(x, sm, precision=jax.lax.Precision.HIGHEST)
        s2 = jax.lax.dot(x * x, sm, precision=jax.lax.Precision.HIGHEST)
        m = s1 * inv_e
        v = s2 * inv_e - m * m
        o_ref[...] = ((x - m) * jax.lax.rsqrt(v + jnp.float32(1e-6))
                      * g_ref[...] + b_ref[...])

    return pl.pallas_call(
        body,
        grid=(n_rows // blk,),
        in_specs=[
            pl.BlockSpec((blk, TCP), lambda i: (i, 0)),
            pl.BlockSpec((TCP, TCP), lambda i: (0, 0)),
            pl.BlockSpec((1, TCP), lambda i: (0, 0)),
            pl.BlockSpec((1, TCP), lambda i: (0, 0)),
        ],
        out_specs=pl.BlockSpec((blk, TCP), lambda i: (i, 0)),
        out_shape=jax.ShapeDtypeStruct((n_rows, TCP), jnp.float32),
    )


def kernel(src, seg, word_table, pos_table, gamma, beta):
    b, l = src.shape
    vocab, emb = word_table.shape
    n_pos = pos_table.shape[0]
    n = b * l
    assert TCP % emb == 0 and (n * emb) % TCP == 0
    flat_src = src.reshape(n).astype(jnp.int32)
    flat_seg = seg.reshape(n).astype(jnp.int32)

    sc = _make_sc_gather(n, vocab, emb, n_pos)
    rows = sc(flat_src, flat_seg, word_table, pos_table)

    n_rows = n * emb // TCP
    x2 = rows.reshape(n_rows, TCP)
    seg_ids = jnp.arange(TCP, dtype=jnp.int32) // emb
    segm = (seg_ids[:, None] == seg_ids[None, :]).astype(jnp.float32)
    rep = TCP // emb
    g2 = jnp.tile(gamma, rep).reshape(1, TCP).astype(jnp.float32)
    b2 = jnp.tile(beta, rep).reshape(1, TCP).astype(jnp.float32)

    ln = _make_tc_layernorm(n_rows, emb)
    y = ln(x2, segm, g2, b2)
    return y.reshape(b, l, emb)
